# baseline (speedup 1.0000x reference)
import functools

import jax
import jax.numpy as jnp
from jax import lax
from jax.experimental import pallas as pl
from jax.experimental.pallas import tpu as pltpu

HEADS = 3
NEG_SLOPE = 0.2


def _leaky_relu(x):
    return jnp.where(x >= 0, x, NEG_SLOPE * x)


def _elu(x):
    return jnp.where(x > 0, x, jnp.exp(x) - 1.0)


def _gat_dense(h, a_src_row, a_dst_row):
    d = h.shape[1]
    ad_col = jnp.sum(h * a_dst_row, axis=1, keepdims=True)
    as_row = lax.dot_general(a_src_row, h, (((1,), (1,)), ((), ())),
                             preferred_element_type=jnp.float32)
    as_max = jnp.max(as_row, axis=1, keepdims=True)
    e = _leaky_relu(ad_col + as_row)
    emax = _leaky_relu(ad_col + as_max)
    ee = jnp.exp(e - emax)
    h_aug = jnp.concatenate([h, jnp.ones_like(h[:, :1])], axis=1)
    agg = jnp.dot(ee, h_aug, preferred_element_type=jnp.float32)
    return agg[:, :d] / (agg[:, d:d + 1] + 1e-16)


def _gnn_kernel(xt_ref, mask_ref, w1t_ref, a1s_ref, a1d_ref, w2t_hbm,
                a2s_ref, a2d_ref, p1w_hbm, p1b_ref, p2wt_ref, p2b_ref,
                out_ref, mask_out_ref, w2t_v, p1w_v, s_w2, s_p1w,
                *, n, jobs, batch, h1dim):
    c_w2 = pltpu.make_async_copy(w2t_hbm, w2t_v, s_w2)
    c_p1w = pltpu.make_async_copy(p1w_hbm, p1w_v, s_p1w)
    c_w2.start()
    c_p1w.start()

    xt = xt_ref[...]
    h_all = lax.dot_general(xt, w1t_ref[...], (((0,), (0,)), ((), ())),
                            preferred_element_type=jnp.float32)
    head_outs = []
    for h in range(HEADS):
        hfeat = h_all[:, h * h1dim:(h + 1) * h1dim]
        a_s = a1s_ref[pl.ds(h, 1), :]
        a_d = a1d_ref[pl.ds(h, 1), :]
        head_outs.append(_gat_dense(hfeat, a_s, a_d))
    h1 = _elu(jnp.concatenate(head_outs, axis=1))

    c_w2.wait()
    h2feat = lax.dot_general(h1, w2t_v[...], (((1,), (1,)), ((), ())),
                             preferred_element_type=jnp.float32)
    h2 = _elu(_gat_dense(h2feat, a2s_ref[...], a2d_ref[...]))

    c_p1w.wait()
    f1 = jnp.maximum(
        jnp.dot(h2, p1w_v[...], preferred_element_type=jnp.float32)
        + p1b_ref[...], 0.0)
    f2 = jnp.maximum(
        lax.dot_general(f1, p2wt_ref[...], (((1,), (1,)), ((), ())),
                        preferred_element_type=jnp.float32)
        + p2b_ref[...], 0.0)

    row_b = lax.broadcasted_iota(jnp.int32, (batch, n), 0)
    col_n = lax.broadcasted_iota(jnp.int32, (batch, n), 1)
    lo = row_b * jobs
    sel = jnp.where((col_n >= lo) & (col_n < lo + jobs), 1.0 / jobs, 0.0)
    out_ref[...] = jnp.dot(sel, f2, preferred_element_type=jnp.float32)
    mask_out_ref[...] = mask_ref[...]


@jax.jit
def kernel(real_obs, action_mask, W1, a1_src, a1_dst, W2, a2_src, a2_dst,
           P1w, P1b, P2w, P2b):
    B, J, F = real_obs.shape
    N = B * J
    H1 = W1.shape[2]
    OUT2 = W2.shape[1]
    HID = P2w.shape[1]

    xt = real_obs.reshape(N, F).T
    w1t = W1.transpose(1, 0, 2).reshape(F, HEADS * H1)
    w2t = W2.T
    p2wt = P2w.T

    vspec = pl.BlockSpec(memory_space=pltpu.VMEM)
    aspec = pl.BlockSpec(memory_space=pl.ANY)
    body = functools.partial(_gnn_kernel, n=N, jobs=J, batch=B, h1dim=H1)
    feats, mask_out = pl.pallas_call(
        body,
        in_specs=[vspec, vspec, vspec, vspec, vspec, aspec, vspec, vspec,
                  aspec, vspec, vspec, vspec],
        out_shape=(jax.ShapeDtypeStruct((B, HID), jnp.float32),
                   jax.ShapeDtypeStruct((B, J), action_mask.dtype)),
        scratch_shapes=[pltpu.VMEM((OUT2, HEADS * H1), jnp.float32),
                        pltpu.VMEM((OUT2, 2 * HID), jnp.float32),
                        pltpu.SemaphoreType.DMA, pltpu.SemaphoreType.DMA],
    )(xt, action_mask, w1t, a1_src, a1_dst, w2t,
      a2_src.reshape(1, -1), a2_dst.reshape(1, -1),
      P1w, P1b.reshape(1, -1), p2wt, P2b.reshape(1, -1))
    return feats, mask_out
